# Initial kernel scaffold; baseline (speedup 1.0000x reference)
import jax
import jax.numpy as jnp
from jax.experimental import pallas as pl


def kernel(node_emb, edge_emb, edge_index):
    raise NotImplementedError("write your pallas kernel here")


# speedup vs baseline: 2.3201x; 2.3201x over previous
import functools

import jax
import jax.numpy as jnp
from jax import lax
from jax.experimental import pallas as pl
from jax.experimental.pallas import tpu as pltpu
from jax.experimental.pallas import tpu_sc as plsc

N_NODES = 10000
N_EDGES = 320000
D = 128
L = 16
NC = 2
NS = 16
NW = NC * NS
EPW = N_EDGES // NW
C = 80
NCHUNK = EPW // C
GRP = C // L


def _dist_mult_body(src_hbm, dst_hbm, node_hbm, edge_hbm, out_hbm,
                    idx_s, idx_d, head_v, tail_v, rel_v, scores_v,
                    sem_h, sem_t):
    wid = lax.axis_index("c") * NS + lax.axis_index("s")
    base = wid * EPW

    def chunk_body(c, _):
        off = base + c * C
        pltpu.sync_copy(src_hbm.at[pl.ds(off, C)], idx_s)
        pltpu.sync_copy(dst_hbm.at[pl.ds(off, C)], idx_d)
        cp_h = pltpu.async_copy(node_hbm.at[idx_s], head_v, sem_h)
        cp_t = pltpu.async_copy(node_hbm.at[idx_d], tail_v, sem_t)
        pltpu.sync_copy(edge_hbm.at[pl.ds(off, C)], rel_v)
        cp_h.wait()
        cp_t.wait()

        lane = lax.iota(jnp.int32, L)

        def grp_body(g, _):
            svec = jnp.zeros((L,), jnp.float32)
            for i in range(L):
                e = g * L + i
                acc = jnp.zeros((L,), jnp.float32)
                for j in range(D // L):
                    sl = pl.ds(j * L, L)
                    acc = acc + head_v[e, sl] * rel_v[e, sl] * tail_v[e, sl]
                svec = jnp.where(lane == i, jnp.sum(acc), svec)
            scores_v[pl.ds(g * L, L)] = svec
            return 0

        lax.fori_loop(0, GRP, grp_body, 0)
        pltpu.sync_copy(scores_v, out_hbm.at[pl.ds(off, C)])
        return 0

    lax.fori_loop(0, NCHUNK, chunk_body, 0)


@jax.jit
def _dist_mult(src_idx, dst_idx, node_emb, edge_emb):
    mesh = plsc.VectorSubcoreMesh(
        core_axis_name="c", subcore_axis_name="s",
        num_cores=NC, num_subcores=NS)
    return pl.kernel(
        _dist_mult_body,
        out_type=jax.ShapeDtypeStruct((N_EDGES,), jnp.float32),
        mesh=mesh,
        scratch_types=[
            pltpu.VMEM((C,), jnp.int32),
            pltpu.VMEM((C,), jnp.int32),
            pltpu.VMEM((C, D), jnp.float32),
            pltpu.VMEM((C, D), jnp.float32),
            pltpu.VMEM((C, D), jnp.float32),
            pltpu.VMEM((C,), jnp.float32),
            pltpu.SemaphoreType.DMA,
            pltpu.SemaphoreType.DMA,
        ],
        compiler_params=pltpu.CompilerParams(needs_layout_passes=False),
    )(src_idx, dst_idx, node_emb, edge_emb)


def kernel(node_emb, edge_emb, edge_index):
    src = edge_index[0].astype(jnp.int32)
    dst = edge_index[1].astype(jnp.int32)
    return _dist_mult(src, dst, node_emb, edge_emb)


# speedup vs baseline: 2.6754x; 1.1531x over previous
import functools

import jax
import jax.numpy as jnp
from jax import lax
from jax.experimental import pallas as pl
from jax.experimental.pallas import tpu as pltpu
from jax.experimental.pallas import tpu_sc as plsc

N_NODES = 10000
N_EDGES = 320000
D = 128
L = 16
NC = 2
NS = 16
NW = NC * NS
EPW = N_EDGES // NW
C = 80
NCHUNK = EPW // C
GRP = C // L


def _dist_mult_body(src_hbm, dst_hbm, node_hbm, edge_hbm, out_hbm,
                    idx_s, idx_d, scores_all,
                    head0, tail0, rel0, head1, tail1, rel1,
                    sh0, st0, sr0, sh1, st1, sr1):
    wid = lax.axis_index("c") * NS + lax.axis_index("s")
    base = wid * EPW

    pltpu.sync_copy(src_hbm.at[pl.ds(base, EPW)], idx_s)
    pltpu.sync_copy(dst_hbm.at[pl.ds(base, EPW)], idx_d)

    bufs = ((head0, tail0, rel0, sh0, st0, sr0),
            (head1, tail1, rel1, sh1, st1, sr1))

    def fire(c, b):
        head_v, tail_v, rel_v, sh, st, sr = bufs[b]
        off = c * C
        pltpu.async_copy(node_hbm.at[idx_s.at[pl.ds(off, C)]], head_v, sh)
        pltpu.async_copy(node_hbm.at[idx_d.at[pl.ds(off, C)]], tail_v, st)
        pltpu.async_copy(edge_hbm.at[pl.ds(base + off, C)], rel_v, sr)

    def wait(c, b):
        head_v, tail_v, rel_v, sh, st, sr = bufs[b]
        off = c * C
        pltpu.make_async_copy(
            node_hbm.at[idx_s.at[pl.ds(off, C)]], head_v, sh).wait()
        pltpu.make_async_copy(
            node_hbm.at[idx_d.at[pl.ds(off, C)]], tail_v, st).wait()
        pltpu.make_async_copy(
            edge_hbm.at[pl.ds(base + off, C)], rel_v, sr).wait()

    lane = lax.iota(jnp.int32, L)

    def compute(c, b):
        head_v, tail_v, rel_v = bufs[b][:3]

        def grp_body(g, _):
            svec = jnp.zeros((L,), jnp.float32)
            for i in range(L):
                e = g * L + i
                acc = jnp.zeros((L,), jnp.float32)
                for j in range(D // L):
                    sl = pl.ds(j * L, L)
                    acc = acc + head_v[e, sl] * rel_v[e, sl] * tail_v[e, sl]
                svec = jnp.where(lane == i, jnp.sum(acc), svec)
            scores_all[pl.ds(c * C + g * L, L)] = svec
            return 0

        lax.fori_loop(0, GRP, grp_body, 0)

    fire(0, 0)

    def pair_body(k, _):
        c0 = 2 * k
        fire(c0 + 1, 1)
        wait(c0, 0)
        compute(c0, 0)
        fire(c0 + 2, 0)
        wait(c0 + 1, 1)
        compute(c0 + 1, 1)
        return 0

    lax.fori_loop(0, (NCHUNK - 1) // 2, pair_body, 0)
    wait(NCHUNK - 1, 0)
    compute(NCHUNK - 1, 0)

    pltpu.sync_copy(scores_all, out_hbm.at[pl.ds(base, EPW)])


@jax.jit
def _dist_mult(src_idx, dst_idx, node_emb, edge_emb):
    mesh = plsc.VectorSubcoreMesh(
        core_axis_name="c", subcore_axis_name="s",
        num_cores=NC, num_subcores=NS)
    return pl.kernel(
        _dist_mult_body,
        out_type=jax.ShapeDtypeStruct((N_EDGES,), jnp.float32),
        mesh=mesh,
        scratch_types=[
            pltpu.VMEM((EPW,), jnp.int32),
            pltpu.VMEM((EPW,), jnp.int32),
            pltpu.VMEM((EPW,), jnp.float32),
            pltpu.VMEM((C, D), jnp.float32),
            pltpu.VMEM((C, D), jnp.float32),
            pltpu.VMEM((C, D), jnp.float32),
            pltpu.VMEM((C, D), jnp.float32),
            pltpu.VMEM((C, D), jnp.float32),
            pltpu.VMEM((C, D), jnp.float32),
            pltpu.SemaphoreType.DMA,
            pltpu.SemaphoreType.DMA,
            pltpu.SemaphoreType.DMA,
            pltpu.SemaphoreType.DMA,
            pltpu.SemaphoreType.DMA,
            pltpu.SemaphoreType.DMA,
        ],
        compiler_params=pltpu.CompilerParams(needs_layout_passes=False),
    )(src_idx, dst_idx, node_emb, edge_emb)


def kernel(node_emb, edge_emb, edge_index):
    src = edge_index[0].astype(jnp.int32)
    dst = edge_index[1].astype(jnp.int32)
    return _dist_mult(src, dst, node_emb, edge_emb)
